# baseline (speedup 1.0000x reference)
import functools

import jax
import jax.numpy as jnp
from jax import lax
from jax.experimental import pallas as pl
from jax.experimental.pallas import tpu as pltpu
from jax.experimental.pallas import tpu_sc as plsc

LAM = 0.1
K_ITERS = 3
_GAMMA = 1.0 / (2.0 * (1.0 - LAM))
_THR = _GAMMA * LAM
_G = _GAMMA * 2.0 * (1.0 - LAM)

_NSUB = 16
_NCORES = 2
_EB = 128


def _ceil_to(a, m):
    return (a + m - 1) // m * m


def _deg_body(npad, nb, col32_h, zeros_h, ones_h, deg2_h,
              col_v, ones_v, deg_sh, sem):
    c = lax.axis_index("c")
    s = lax.axis_index("s")
    t = c * _NSUB + s
    rows = npad // _NSUB
    pltpu.sync_copy(zeros_h.at[pl.ds(s * rows, rows)],
                    deg_sh.at[pl.ds(s * rows, rows)])
    pltpu.sync_copy(col32_h.at[t], col_v)
    pltpu.sync_copy(ones_h, ones_v)
    plsc.subcore_barrier()

    @pl.loop(0, nb)
    def _(j):
        pltpu.sync_copy(ones_v, deg_sh.at[col_v.at[j]], add=True)

    plsc.subcore_barrier()
    pltpu.sync_copy(deg_sh.at[pl.ds(s * rows, rows)],
                    deg2_h.at[c].at[pl.ds(s * rows, rows)])


def _make_deg(n, npad, epad):
    nb = epad // (_NCORES * _NSUB) // _EB
    mesh = plsc.VectorSubcoreMesh(core_axis_name="c", subcore_axis_name="s")
    return pl.kernel(
        functools.partial(_deg_body, npad, nb),
        out_type=jax.ShapeDtypeStruct((_NCORES, npad, 128), jnp.float32),
        mesh=mesh,
        scratch_types=[
            pltpu.VMEM((nb, _EB), jnp.int32),
            pltpu.VMEM((_EB, 128), jnp.float32),
            pltpu.VMEM_SHARED((npad, 128), jnp.float32),
            pltpu.SemaphoreType.DMA,
        ],
    )


_BS = 80
_NSLOT = 4
_NBS = 4


def _unpack_idx(pk_v, cb, j, idx_v, slot):
    for i in range(_BS // 16):
        pk = pk_v[cb, j, pl.ds(i * 16, 16)]
        idx_v[slot, 0, pl.ds(i * 16, 16)] = lax.bitwise_and(pk, 0xFFFF)
        idx_v[slot, 1, pl.ds(i * 16, 16)] = lax.shift_right_logical(pk, 16)


def _spmm_body(n, npad, nsteps, z2_h, pk16_h, u2_h,
               pk_v, idx_v, gbuf, u_sh, sem):
    c = lax.axis_index("c")
    s = lax.axis_index("s")
    base = (n // _NSUB) // 8 * 8
    rem = n - base * _NSUB
    pltpu.sync_copy(z2_h.at[c].at[pl.ds(s * base, base)],
                    u_sh.at[pl.ds(s * base, base)])
    if rem:
        @pl.when(s == _NSUB - 1)
        def _():
            pltpu.sync_copy(z2_h.at[c].at[pl.ds(base * _NSUB, rem)],
                            u_sh.at[pl.ds(base * _NSUB, rem)])
    pltpu.sync_copy(pk16_h.at[s, 0], pk_v.at[0])
    plsc.subcore_barrier()
    zc = z2_h.at[c]
    gs = sem[:_NSLOT]
    ss = sem[_NSLOT:2 * _NSLOT]
    pks = sem[2 * _NSLOT]

    @pl.loop(0, nsteps)
    def _(t):
        cb = lax.rem(t, 2)

        @pl.when(t > 0)
        def _():
            pltpu.make_async_copy(pk16_h.at[s, t], pk_v.at[cb], pks).wait()
            for k in range(_NSLOT):
                pltpu.make_async_copy(gbuf.at[k], u_sh.at[idx_v.at[k, 1]],
                                      ss[k]).wait()

        for k in range(_NSLOT):
            _unpack_idx(pk_v, cb, k, idx_v, k)

        @pl.when(t < nsteps - 1)
        def _():
            pltpu.async_copy(pk16_h.at[s, t + 1], pk_v.at[1 - cb], pks)

        dg = [pltpu.async_copy(zc.at[idx_v.at[k, 0]], gbuf.at[k], gs[k])
              for k in range(_NSLOT)]
        for k in range(_NSLOT):
            dg[k].wait()
            pltpu.async_copy(gbuf.at[k], u_sh.at[idx_v.at[k, 1]],
                             ss[k], add=True)

    for k in range(_NSLOT):
        pltpu.make_async_copy(gbuf.at[k], u_sh.at[idx_v.at[k, 1]],
                              ss[k]).wait()
    plsc.subcore_barrier()
    pltpu.sync_copy(u_sh.at[pl.ds(s * base, base)],
                    u2_h.at[c].at[pl.ds(s * base, base)])
    if rem:
        @pl.when(s == _NSUB - 1)
        def _():
            pltpu.sync_copy(u_sh.at[pl.ds(base * _NSUB, rem)],
                            u2_h.at[c].at[pl.ds(base * _NSUB, rem)])


def _make_spmm(n, npad, epad):
    nsteps = epad // _NSUB // (_NBS * _BS)
    mesh = plsc.VectorSubcoreMesh(core_axis_name="c", subcore_axis_name="s")
    return pl.kernel(
        functools.partial(_spmm_body, n, npad, nsteps),
        out_type=jax.ShapeDtypeStruct((_NCORES, npad, 128), jnp.float32),
        mesh=mesh,
        scratch_types=[
            pltpu.VMEM((2, _NBS, _BS), jnp.int32),
            pltpu.VMEM((_NSLOT, 2, _BS), jnp.int32),
            pltpu.VMEM((_NSLOT, _BS, 128), jnp.float32),
            pltpu.VMEM_SHARED((npad, 128), jnp.float32),
            tuple(pltpu.SemaphoreType.DMA for _ in range(2 * _NSLOT + 1)),
        ],
    )


def _prep_body(deg2_ref, x_ref, dinv_ref, z2_ref):
    deg = deg2_ref[0, :, 0:1] + deg2_ref[1, :, 0:1] + 1.0
    dinv = lax.rsqrt(deg)
    dinv_ref[...] = dinv
    xv = x_ref[...]
    z2_ref[0] = dinv * xv[:, :128]
    z2_ref[1] = dinv * xv[:, 128:]


def _make_prep(n, npad, nb):
    grid = n // nb
    return pl.pallas_call(
        _prep_body,
        grid=(grid,),
        in_specs=[
            pl.BlockSpec((_NCORES, nb, 128), lambda i: (0, i, 0)),
            pl.BlockSpec((nb, 256), lambda i: (i, 0)),
        ],
        out_specs=[
            pl.BlockSpec((nb, 1), lambda i: (i, 0)),
            pl.BlockSpec((_NCORES, nb, 128), lambda i: (0, i, 0)),
        ],
        out_shape=[
            jax.ShapeDtypeStruct((n, 1), jnp.float32),
            jax.ShapeDtypeStruct((_NCORES, n, 128), jnp.float32),
        ],
    )


def _post_body(u2_ref, hh_ref, dinv_ref, xo_ref, z2_ref):
    dinv = dinv_ref[...]
    hh = hh_ref[...]
    h0 = hh[:, :128]
    h1 = hh[:, 128:]
    y0 = _G * (dinv * u2_ref[0])
    y1 = _G * (dinv * u2_ref[1])
    d0 = y0 - h0
    d1 = y1 - h1
    rn2 = (jnp.sum(d0 * d0, axis=1, keepdims=True)
           + jnp.sum(d1 * d1, axis=1, keepdims=True))
    rn = jnp.sqrt(rn2)
    score = jnp.maximum(rn - _THR, 0.0)
    sc = jnp.where(rn > 0, score / jnp.where(rn > 0, rn, 1.0), score)
    x0 = h0 + sc * d0
    x1 = h1 + sc * d1
    xo_ref[...] = jnp.concatenate([x0, x1], axis=1)
    z2_ref[0] = dinv * x0
    z2_ref[1] = dinv * x1


def _make_post(n, npad, nb):
    grid = n // nb
    return pl.pallas_call(
        _post_body,
        grid=(grid,),
        in_specs=[
            pl.BlockSpec((_NCORES, nb, 128), lambda i: (0, i, 0)),
            pl.BlockSpec((nb, 256), lambda i: (i, 0)),
            pl.BlockSpec((nb, 1), lambda i: (i, 0)),
        ],
        out_specs=[
            pl.BlockSpec((nb, 256), lambda i: (i, 0)),
            pl.BlockSpec((_NCORES, nb, 128), lambda i: (0, i, 0)),
        ],
        out_shape=[
            jax.ShapeDtypeStruct((n, 256), jnp.float32),
            jax.ShapeDtypeStruct((_NCORES, n, 128), jnp.float32),
        ],
    )


def kernel(x, edge_index):
    n, d = x.shape
    e = edge_index.shape[1]
    assert d == 256 and n % _NSUB == 0 and n % 8 == 0

    epad = _ceil_to(e, 20480)
    npad = _ceil_to(n + 1, 128)

    row = edge_index[0]
    col = edge_index[1]
    pad = epad - e
    if pad:
        row = jnp.concatenate([row, jnp.zeros((pad,), row.dtype)])
        col = jnp.concatenate([col, jnp.full((pad,), n, col.dtype)])
    pk16 = (row | (col << 16)).reshape(_NSUB, -1, _NBS, _BS)
    col32 = col.reshape(_NCORES * _NSUB, -1, _EB)
    ones_h = jnp.ones((_EB, 128), jnp.float32)
    zeros_h = jnp.zeros((npad, 128), jnp.float32)

    deg2 = _make_deg(n, npad, epad)(col32, zeros_h, ones_h)

    nb_tc = 2000
    dinv, z2 = _make_prep(n, npad, nb_tc)(deg2, x)
    spmm = _make_spmm(n, npad, epad)
    post = _make_post(n, npad, nb_tc)
    xo = x
    for _ in range(K_ITERS):
        u2 = spmm(z2, pk16)
        xo, z2 = post(u2, x, dinv)
    return xo


# speedup vs baseline: 1.0328x; 1.0328x over previous
import functools

import jax
import jax.numpy as jnp
from jax import lax
from jax.experimental import pallas as pl
from jax.experimental.pallas import tpu as pltpu
from jax.experimental.pallas import tpu_sc as plsc

LAM = 0.1
K_ITERS = 3
_GAMMA = 1.0 / (2.0 * (1.0 - LAM))
_THR = _GAMMA * LAM
_G = _GAMMA * 2.0 * (1.0 - LAM)

_NSUB = 16
_NCORES = 2
_EB = 128


def _ceil_to(a, m):
    return (a + m - 1) // m * m


def _deg_body(npad, nb, col32_h, zeros_h, ones_h, deg2_h,
              col_v, ones_v, deg_sh, sem):
    c = lax.axis_index("c")
    s = lax.axis_index("s")
    t = c * _NSUB + s
    rows = npad // _NSUB
    pltpu.sync_copy(zeros_h.at[pl.ds(s * rows, rows)],
                    deg_sh.at[pl.ds(s * rows, rows)])
    pltpu.sync_copy(col32_h.at[t], col_v)
    pltpu.sync_copy(ones_h, ones_v)
    plsc.subcore_barrier()

    @pl.loop(0, nb)
    def _(j):
        pltpu.sync_copy(ones_v, deg_sh.at[col_v.at[j]], add=True)

    plsc.subcore_barrier()
    pltpu.sync_copy(deg_sh.at[pl.ds(s * rows, rows)],
                    deg2_h.at[c].at[pl.ds(s * rows, rows)])


def _make_deg(n, npad, epad):
    nb = epad // (_NCORES * _NSUB) // _EB
    mesh = plsc.VectorSubcoreMesh(core_axis_name="c", subcore_axis_name="s")
    return pl.kernel(
        functools.partial(_deg_body, npad, nb),
        out_type=jax.ShapeDtypeStruct((_NCORES, npad, 128), jnp.float32),
        mesh=mesh,
        scratch_types=[
            pltpu.VMEM((nb, _EB), jnp.int32),
            pltpu.VMEM((_EB, 128), jnp.float32),
            pltpu.VMEM_SHARED((npad, 128), jnp.float32),
            pltpu.SemaphoreType.DMA,
        ],
    )


_BS = 80
_NSLOT = 4
_NBS = 4


def _unpack_idx(pk_v, cb, j, idx_v, bank):
    for i in range(_BS // 16):
        pk = pk_v[cb, j, pl.ds(i * 16, 16)]
        idx_v[bank, j, 0, pl.ds(i * 16, 16)] = lax.bitwise_and(pk, 0xFFFF)
        idx_v[bank, j, 1, pl.ds(i * 16, 16)] = lax.shift_right_logical(pk, 16)


def _spmm_body(n, npad, nsteps, z2_h, pk16_h, u2_h,
               pk_v, idx_v, gbuf, u_sh, sem):
    c = lax.axis_index("c")
    s = lax.axis_index("s")
    base = (n // _NSUB) // 8 * 8
    rem = n - base * _NSUB
    pltpu.sync_copy(z2_h.at[c].at[pl.ds(s * base, base)],
                    u_sh.at[pl.ds(s * base, base)])
    if rem:
        @pl.when(s == _NSUB - 1)
        def _():
            pltpu.sync_copy(z2_h.at[c].at[pl.ds(base * _NSUB, rem)],
                            u_sh.at[pl.ds(base * _NSUB, rem)])
    pltpu.sync_copy(pk16_h.at[s, 0], pk_v.at[0])
    plsc.subcore_barrier()
    for k in range(_NSLOT):
        _unpack_idx(pk_v, 0, k, idx_v, 0)
    zc = z2_h.at[c]
    gsa, gsb, ssa, ssb, pks = sem

    @pl.loop(0, nsteps)
    def _(t):
        cb = lax.rem(t, 2)
        sems = [gsa, gsb, ssa, ssb]
        dg = [pltpu.async_copy(zc.at[idx_v.at[cb, k, 0]], gbuf.at[k],
                               sems[k % 2])
              for k in range(_NSLOT)]

        @pl.when(t < nsteps - 1)
        def _():
            pltpu.async_copy(pk16_h.at[s, t + 1], pk_v.at[1 - cb], pks)

        ds = []
        for k in range(_NSLOT):
            dg[k].wait()
            ds.append(pltpu.async_copy(gbuf.at[k],
                                       u_sh.at[idx_v.at[cb, k, 1]],
                                       sems[2 + k % 2], add=True))

        @pl.when(t < nsteps - 1)
        def _():
            pltpu.make_async_copy(pk16_h.at[s, t + 1], pk_v.at[1 - cb],
                                  pks).wait()
            for k in range(_NSLOT):
                _unpack_idx(pk_v, 1 - cb, k, idx_v, 1 - cb)

        for d in ds:
            d.wait()

    plsc.subcore_barrier()
    pltpu.sync_copy(u_sh.at[pl.ds(s * base, base)],
                    u2_h.at[c].at[pl.ds(s * base, base)])
    if rem:
        @pl.when(s == _NSUB - 1)
        def _():
            pltpu.sync_copy(u_sh.at[pl.ds(base * _NSUB, rem)],
                            u2_h.at[c].at[pl.ds(base * _NSUB, rem)])


def _make_spmm(n, npad, epad):
    nsteps = epad // _NSUB // (_NBS * _BS)
    mesh = plsc.VectorSubcoreMesh(core_axis_name="c", subcore_axis_name="s")
    return pl.kernel(
        functools.partial(_spmm_body, n, npad, nsteps),
        out_type=jax.ShapeDtypeStruct((_NCORES, npad, 128), jnp.float32),
        mesh=mesh,
        scratch_types=[
            pltpu.VMEM((2, _NBS, _BS), jnp.int32),
            pltpu.VMEM((2, _NSLOT, 2, _BS), jnp.int32),
            pltpu.VMEM((_NSLOT, _BS, 128), jnp.float32),
            pltpu.VMEM_SHARED((npad, 128), jnp.float32),
            (pltpu.SemaphoreType.DMA, pltpu.SemaphoreType.DMA,
             pltpu.SemaphoreType.DMA, pltpu.SemaphoreType.DMA,
             pltpu.SemaphoreType.DMA),
        ],
    )


def _prep_body(deg2_ref, x_ref, dinv_ref, z2_ref):
    deg = deg2_ref[0, :, 0:1] + deg2_ref[1, :, 0:1] + 1.0
    dinv = lax.rsqrt(deg)
    dinv_ref[...] = dinv
    xv = x_ref[...]
    z2_ref[0] = dinv * xv[:, :128]
    z2_ref[1] = dinv * xv[:, 128:]


def _make_prep(n, npad, nb):
    grid = n // nb
    return pl.pallas_call(
        _prep_body,
        grid=(grid,),
        in_specs=[
            pl.BlockSpec((_NCORES, nb, 128), lambda i: (0, i, 0)),
            pl.BlockSpec((nb, 256), lambda i: (i, 0)),
        ],
        out_specs=[
            pl.BlockSpec((nb, 1), lambda i: (i, 0)),
            pl.BlockSpec((_NCORES, nb, 128), lambda i: (0, i, 0)),
        ],
        out_shape=[
            jax.ShapeDtypeStruct((n, 1), jnp.float32),
            jax.ShapeDtypeStruct((_NCORES, n, 128), jnp.float32),
        ],
    )


def _post_body(u2_ref, hh_ref, dinv_ref, xo_ref, z2_ref):
    dinv = dinv_ref[...]
    hh = hh_ref[...]
    h0 = hh[:, :128]
    h1 = hh[:, 128:]
    y0 = _G * (dinv * u2_ref[0])
    y1 = _G * (dinv * u2_ref[1])
    d0 = y0 - h0
    d1 = y1 - h1
    rn2 = (jnp.sum(d0 * d0, axis=1, keepdims=True)
           + jnp.sum(d1 * d1, axis=1, keepdims=True))
    rn = jnp.sqrt(rn2)
    score = jnp.maximum(rn - _THR, 0.0)
    sc = jnp.where(rn > 0, score / jnp.where(rn > 0, rn, 1.0), score)
    x0 = h0 + sc * d0
    x1 = h1 + sc * d1
    xo_ref[...] = jnp.concatenate([x0, x1], axis=1)
    z2_ref[0] = dinv * x0
    z2_ref[1] = dinv * x1


def _make_post(n, npad, nb):
    grid = n // nb
    return pl.pallas_call(
        _post_body,
        grid=(grid,),
        in_specs=[
            pl.BlockSpec((_NCORES, nb, 128), lambda i: (0, i, 0)),
            pl.BlockSpec((nb, 256), lambda i: (i, 0)),
            pl.BlockSpec((nb, 1), lambda i: (i, 0)),
        ],
        out_specs=[
            pl.BlockSpec((nb, 256), lambda i: (i, 0)),
            pl.BlockSpec((_NCORES, nb, 128), lambda i: (0, i, 0)),
        ],
        out_shape=[
            jax.ShapeDtypeStruct((n, 256), jnp.float32),
            jax.ShapeDtypeStruct((_NCORES, n, 128), jnp.float32),
        ],
    )


def kernel(x, edge_index):
    n, d = x.shape
    e = edge_index.shape[1]
    assert d == 256 and n % _NSUB == 0 and n % 8 == 0

    epad = _ceil_to(e, 20480)
    npad = _ceil_to(n + 1, 128)

    row = edge_index[0]
    col = edge_index[1]
    pad = epad - e
    if pad:
        row = jnp.concatenate([row, jnp.zeros((pad,), row.dtype)])
        col = jnp.concatenate([col, jnp.full((pad,), n, col.dtype)])
    pk16 = (row | (col << 16)).reshape(_NSUB, -1, _NBS, _BS)
    col32 = col.reshape(_NCORES * _NSUB, -1, _EB)
    ones_h = jnp.ones((_EB, 128), jnp.float32)
    zeros_h = jnp.zeros((npad, 128), jnp.float32)

    deg2 = _make_deg(n, npad, epad)(col32, zeros_h, ones_h)

    nb_tc = 2000
    dinv, z2 = _make_prep(n, npad, nb_tc)(deg2, x)
    spmm = _make_spmm(n, npad, epad)
    post = _make_post(n, npad, nb_tc)
    xo = x
    for _ in range(K_ITERS):
        u2 = spmm(z2, pk16)
        xo, z2 = post(u2, x, dinv)
    return xo
